# baseline (speedup 1.0000x reference)
import functools

import jax
import jax.numpy as jnp
from jax import lax
from jax.experimental import pallas as pl
from jax.experimental.pallas import tpu as pltpu
from jax.experimental.pallas import tpu_sc as plsc

N_LAYERS_ = 3
NUM_TECS = 16
CHUNK_E = 256
NSETS = 4
ILV = plsc.PackFormat.INTERLEAVED


def _propagate(n_pad, nnz_pad, chunks_per_tec, egob, rows2d, cols2, vals):
    n2 = 2 * n_pad
    rows_per_tec = n_pad // NUM_TECS
    drain_chunk = 136
    n_drain = rows_per_tec // drain_chunk
    zrows = 184
    n_zero = rows_per_tec // zrows
    assert zrows <= 2 * CHUNK_E and 3 * drain_chunk <= 2 * CHUNK_E
    assert chunks_per_tec % 4 == 0 and chunks_per_tec >= 8
    n_quads = chunks_per_tec // 4
    f32 = jnp.float32
    i32 = jnp.int32

    mesh = plsc.VectorSubcoreMesh(core_axis_name="c", subcore_axis_name="s")

    @functools.partial(
        pl.kernel,
        mesh=mesh,
        compiler_params=pltpu.CompilerParams(use_tc_tiling_on_sc=False,
                                             needs_layout_passes=False),
        out_type=(
            jax.ShapeDtypeStruct((n2, 32), f32),
            jax.ShapeDtypeStruct((n2, 32), f32),
            jax.ShapeDtypeStruct((n2, 32), f32),
            jax.ShapeDtypeStruct((n2, 16), i32),
            jax.ShapeDtypeStruct((n2, 16), i32),
        ),
        scratch_types=[
            pltpu.VMEM_SHARED((n_pad, 32), f32),
            pltpu.VMEM((2 * CHUNK_E, 32), f32),
            pltpu.VMEM((2 * CHUNK_E, 16), i32),
            pltpu.VMEM((NSETS * CHUNK_E,), i32),
            pltpu.VMEM((NSETS * CHUNK_E,), f32),
            pltpu.VMEM((NSETS, CHUNK_E), i32),
            pltpu.SemaphoreType.DMA((NSETS,)),
            pltpu.SemaphoreType.DMA((2,)),
            pltpu.SemaphoreType.DMA((2,)),
        ],
    )
    def body(egob_h, rows_h, cols_h, vals_h,
             out_h, l1_h, l2_h, l1b_h, l2b_h,
             acc, gbuf, gb16, colv, valv, rowv, lsem, gsem, ssem):
        cid = lax.axis_index("c")
        tid = lax.axis_index("s")
        cid_off = cid * n_pad
        row0 = tid * rows_per_tec
        zeros16 = jnp.zeros((16,), f32)

        def loads_descs(s4, c):
            eb = (tid * chunks_per_tec + c) * CHUNK_E
            return (
                pltpu.make_async_copy(
                    rows_h.at[tid * chunks_per_tec + c],
                    rowv.at[s4],
                    lsem.at[s4]),
                pltpu.make_async_copy(
                    cols_h.at[cid, pl.ds(eb, CHUNK_E)],
                    colv.at[pl.ds(s4 * CHUNK_E, CHUNK_E)],
                    lsem.at[s4]),
                pltpu.make_async_copy(
                    vals_h.at[pl.ds(eb, CHUNK_E)],
                    valv.at[pl.ds(s4 * CHUNK_E, CHUNK_E)],
                    lsem.at[s4]),
            )

        def gather_descs(table_h, s4, g2):
            return (
                pltpu.make_async_copy(
                    table_h.at[colv.at[pl.ds(s4 * CHUNK_E, CHUNK_E)]],
                    gb16.at[pl.ds(g2 * CHUNK_E, CHUNK_E)],
                    gsem.at[g2]),
            )

        def scatter_descs(s4, g2):
            return (
                pltpu.make_async_copy(
                    gbuf.at[pl.ds(g2 * CHUNK_E, CHUNK_E)],
                    acc.at[rowv.at[s4]],
                    ssem.at[g2]),
            )

        def multiply(s4, g2):
            def mb(g, _):
                vv = valv[pl.ds(s4 * CHUNK_E + g * 16, 16)]
                base = g2 * CHUNK_E + g * 16
                for e in range(16):
                    bvec = vv.at[jnp.full((16,), e, i32)].get(
                        mode="promise_in_bounds")
                    gi = gb16[base + e, pl.ds(0, 16)]
                    a = plsc.bitcast(lax.shift_left(gi, 16), f32)
                    b = plsc.bitcast(gi & jnp.int32(-65536), f32)
                    gbuf[base + e, pl.ds(0, 16)] = a * bvec
                    gbuf[base + e, pl.ds(16, 16)] = b * bvec
                return _
            lax.fori_loop(0, CHUNK_E // 16, mb, None)

        def process_edges(table_h):
            for d in loads_descs(0, 0):
                d.start()
            for d in loads_descs(1, 1):
                d.start()

            def quad(q, _):
                for i in range(4):
                    c = 4 * q + i
                    g2 = i % 2
                    s4m, g2m = (i - 1) % 4, (i - 1) % 2

                    for d in loads_descs(i, c):
                        d.wait()

                    def w2(i=i, g2=g2):
                        for d in scatter_descs((i - 2) % 4, g2):
                            d.wait()
                    if i >= 2:
                        w2()
                    else:
                        pl.when(q >= 1)(w2)

                    for d in gather_descs(table_h, i, g2):
                        d.start()

                    def p4(s4m=s4m, g2m=g2m):
                        for d in gather_descs(table_h, s4m, g2m):
                            d.wait()
                        multiply(s4m, g2m)
                        for d in scatter_descs(s4m, g2m):
                            d.start(add=True)
                    if i >= 1:
                        p4()
                    else:
                        pl.when(q >= 1)(p4)

                    def l5(c=c, i=i):
                        for d in loads_descs((i + 2) % 4, c + 2):
                            d.start()
                    if i < 2:
                        l5()
                    else:
                        pl.when(q < n_quads - 1)(l5)
                return _

            lax.fori_loop(0, n_quads, quad, None)

            for d in gather_descs(table_h, 3, 1):
                d.wait()
            multiply(3, 1)
            for d in scatter_descs(3, 1):
                d.start(add=True)
            for d in scatter_descs(2, 0):
                d.wait()
            for d in scatter_descs(3, 1):
                d.wait()

        def drain_packed(dst_b):
            def dbody(ch, _):
                off = row0 + ch * drain_chunk
                pltpu.sync_copy(acc.at[pl.ds(off, drain_chunk)],
                                gbuf.at[pl.ds(0, drain_chunk)])

                def rw(r, _):
                    half = jnp.int32(32768)
                    ai = plsc.bitcast(gbuf[r, pl.ds(0, 16)], i32) + half
                    bi = plsc.bitcast(gbuf[r, pl.ds(16, 16)], i32) + half
                    lo16 = lax.shift_right_logical(ai, 16)
                    hi16 = bi & jnp.int32(-65536)
                    gb16[r, pl.ds(0, 16)] = hi16 | lo16
                    return _
                lax.fori_loop(0, drain_chunk, rw, None)
                pltpu.sync_copy(gb16.at[pl.ds(0, drain_chunk)],
                                dst_b.at[pl.ds(cid_off + off, drain_chunk)])
                return _
            lax.fori_loop(0, n_drain, dbody, None)

        tables = (egob_h, l1b_h, l2b_h)
        for layer in range(N_LAYERS_):
            def zb(i, _):
                gbuf[i, pl.ds(0, 16)] = zeros16
                gbuf[i, pl.ds(16, 16)] = zeros16
                return _
            lax.fori_loop(0, zrows, zb, None)
            for z in range(n_zero):
                pltpu.sync_copy(gbuf.at[pl.ds(0, zrows)],
                                acc.at[pl.ds(row0 + z * zrows, zrows)])
            plsc.subcore_barrier()

            process_edges(tables[layer])
            plsc.subcore_barrier()

            if layer < N_LAYERS_ - 1:
                dst_f = l1_h if layer == 0 else l2_h
                pltpu.sync_copy(
                    acc.at[pl.ds(row0, rows_per_tec)],
                    dst_f.at[pl.ds(cid_off + row0, rows_per_tec)],
                )
                drain_packed(l1b_h if layer == 0 else l2b_h)
            else:
                third = f32(1.0 / 3.0)
                db_o = drain_chunk
                dc_o = 2 * drain_chunk

                def dr(ch, _):
                    off = row0 + ch * drain_chunk
                    pltpu.sync_copy(acc.at[pl.ds(off, drain_chunk)],
                                    gbuf.at[pl.ds(0, drain_chunk)])
                    pltpu.sync_copy(l1_h.at[pl.ds(cid_off + off, drain_chunk)],
                                    gbuf.at[pl.ds(db_o, drain_chunk)])
                    pltpu.sync_copy(l2_h.at[pl.ds(cid_off + off, drain_chunk)],
                                    gbuf.at[pl.ds(dc_o, drain_chunk)])

                    def mr(r, _):
                        for h in (0, 16):
                            s = (gbuf[r, pl.ds(h, 16)]
                                 + gbuf[db_o + r, pl.ds(h, 16)]
                                 + gbuf[dc_o + r, pl.ds(h, 16)]) * third
                            gbuf[r, pl.ds(h, 16)] = s
                        return _
                    lax.fori_loop(0, drain_chunk, mr, None)
                    pltpu.sync_copy(gbuf.at[pl.ds(0, drain_chunk)],
                                    out_h.at[pl.ds(cid_off + off, drain_chunk)])
                    return _
                lax.fori_loop(0, n_drain, dr, None)
            plsc.subcore_barrier()

    return body(egob, rows2d, cols2, vals)


def kernel(user_emb, item_emb, adj_row, adj_col, adj_val):
    n_user = user_emb.shape[0]
    n_item = item_emb.shape[0]
    n_nodes = n_user + n_item
    n_pad = -(-n_nodes // (NUM_TECS * 8)) * (NUM_TECS * 8)
    nnz = adj_row.shape[0]

    quant = NUM_TECS * CHUNK_E * 4
    nnz_pad = -(-nnz // quant) * quant
    chunks_per_tec = nnz_pad // (NUM_TECS * CHUNK_E)
    pad = nnz_pad - nnz
    rows_p = jnp.pad(adj_row, (0, pad))
    cols_p = jnp.pad(adj_col, (0, pad))
    vals_p = jnp.pad(adj_val, (0, pad))
    rows2d = rows_p.reshape(nnz_pad // CHUNK_E, CHUNK_E)
    cols2 = jnp.stack([cols_p, cols_p + jnp.int32(n_pad)])

    ego = jnp.concatenate([user_emb, item_emb], axis=0)
    rpad = n_pad - n_nodes
    lo = jnp.pad(ego[:, :32], ((0, rpad), (0, 0)))
    hi = jnp.pad(ego[:, 32:], ((0, rpad), (0, 0)))
    egof = jnp.concatenate([lo, hi], axis=0)
    inter = jnp.stack([egof[:, :16], egof[:, 16:]], axis=-1).reshape(-1, 32)
    egob = jax.lax.bitcast_convert_type(
        inter.astype(jnp.bfloat16).reshape(-1, 16, 2), jnp.int32)

    out, _l1, _l2, _l1b, _l2b = _propagate(
        n_pad, nnz_pad, chunks_per_tec, egob, rows2d, cols2, vals_p)
    all_emb = jnp.concatenate([out[:n_nodes], out[n_pad:n_pad + n_nodes]],
                              axis=1)
    return (all_emb[:n_user], all_emb[n_user:])


# speedup vs baseline: 1.2553x; 1.2553x over previous
import functools

import jax
import jax.numpy as jnp
from jax import lax
from jax.experimental import pallas as pl
from jax.experimental.pallas import tpu as pltpu
from jax.experimental.pallas import tpu_sc as plsc

N_LAYERS_ = 3
NUM_TECS = 16
CHUNK_E = 192
NSETS = 4
RSETS = 8


def _propagate(n_pad, nnz_pad, chunks_per_tec, ego0, rows2d, cols2, vals):
    n2 = 2 * n_pad
    rows_per_tec = n_pad // NUM_TECS
    drain_chunk = 136
    n_drain = rows_per_tec // drain_chunk
    zrows = 184
    n_zero = rows_per_tec // zrows
    assert zrows <= NSETS * CHUNK_E and 3 * drain_chunk <= NSETS * CHUNK_E
    assert chunks_per_tec % 8 == 0 and chunks_per_tec >= 16
    n_octs = chunks_per_tec // 8
    f32 = jnp.float32
    i32 = jnp.int32

    mesh = plsc.VectorSubcoreMesh(core_axis_name="c", subcore_axis_name="s")

    @functools.partial(
        pl.kernel,
        mesh=mesh,
        compiler_params=pltpu.CompilerParams(use_tc_tiling_on_sc=False),
        out_type=(
            jax.ShapeDtypeStruct((n2, 32), f32),
            jax.ShapeDtypeStruct((n2, 32), f32),
            jax.ShapeDtypeStruct((n2, 32), f32),
        ),
        scratch_types=[
            pltpu.VMEM_SHARED((n_pad, 32), f32),
            pltpu.VMEM((NSETS * CHUNK_E, 32), f32),
            pltpu.VMEM((NSETS * CHUNK_E,), i32),
            pltpu.VMEM((NSETS * CHUNK_E,), f32),
            pltpu.VMEM((RSETS, CHUNK_E), i32),
            pltpu.SemaphoreType.DMA((RSETS,)),
            pltpu.SemaphoreType.DMA((NSETS,)),
            pltpu.SemaphoreType.DMA((NSETS,)),
        ],
    )
    def body(ego_h, rows_h, cols_h, vals_h, out_h, l1_h, l2_h,
             acc, gbuf, colv, valv, rowv, lsem, gsem, ssem):
        cid = lax.axis_index("c")
        tid = lax.axis_index("s")
        cid_off = cid * n_pad
        row0 = tid * rows_per_tec
        zeros16 = jnp.zeros((16,), f32)

        def loads_descs(s8, s4, c):
            eb = (tid * chunks_per_tec + c) * CHUNK_E
            return (
                pltpu.make_async_copy(
                    rows_h.at[tid * chunks_per_tec + c],
                    rowv.at[s8],
                    lsem.at[s8]),
                pltpu.make_async_copy(
                    cols_h.at[cid, pl.ds(eb, CHUNK_E)],
                    colv.at[pl.ds(s4 * CHUNK_E, CHUNK_E)],
                    lsem.at[s8]),
                pltpu.make_async_copy(
                    vals_h.at[pl.ds(eb, CHUNK_E)],
                    valv.at[pl.ds(s4 * CHUNK_E, CHUNK_E)],
                    lsem.at[s8]),
            )

        def gather_desc(table_h, s4):
            return pltpu.make_async_copy(
                table_h.at[colv.at[pl.ds(s4 * CHUNK_E, CHUNK_E)]],
                gbuf.at[pl.ds(s4 * CHUNK_E, CHUNK_E)],
                gsem.at[s4])

        def scatter_desc(s8, s4):
            return pltpu.make_async_copy(
                gbuf.at[pl.ds(s4 * CHUNK_E, CHUNK_E)],
                acc.at[rowv.at[s8]],
                ssem.at[s4])

        def multiply(s4):
            def mb(g, _):
                vv = valv[pl.ds(s4 * CHUNK_E + g * 16, 16)]
                base = s4 * CHUNK_E + g * 16
                for e in range(16):
                    bvec = vv.at[jnp.full((16,), e, i32)].get(
                        mode="promise_in_bounds")
                    gbuf[base + e, pl.ds(0, 16)] = gbuf[base + e, pl.ds(0, 16)] * bvec
                    gbuf[base + e, pl.ds(16, 16)] = gbuf[base + e, pl.ds(16, 16)] * bvec
                return _
            lax.fori_loop(0, CHUNK_E // 16, mb, None)

        def process_edges(table_h):
            for d in loads_descs(0, 0, 0):
                d.start()
            for d in loads_descs(1, 1, 1):
                d.start()

            def oct_(q, _):
                for i in range(8):
                    c = 8 * q + i
                    i4 = i % 4

                    def wsc(i=i, i4=i4):
                        scatter_desc((i - 4) % 8, i4).wait()
                    if i >= 4:
                        wsc()
                    else:
                        pl.when(q >= 1)(wsc)

                    for d in loads_descs(i, i4, c):
                        d.wait()
                    gather_desc(table_h, i4).start()

                    def proc(i=i):
                        gather_desc(table_h, (i - 2) % 4).wait()
                        multiply((i - 2) % 4)
                        scatter_desc((i - 2) % 8, (i - 2) % 4).start(add=True)
                    if i >= 2:
                        proc()
                    else:
                        pl.when(q >= 1)(proc)

                    def l5(c=c, i=i):
                        for d in loads_descs((i + 2) % 8, (i + 2) % 4, c + 2):
                            d.start()
                    if i < 6:
                        l5()
                    else:
                        pl.when(q < n_octs - 1)(l5)
                return _

            lax.fori_loop(0, n_octs, oct_, None)

            for s8m in (6, 7):
                gather_desc(table_h, s8m % 4).wait()
                multiply(s8m % 4)
                scatter_desc(s8m, s8m % 4).start(add=True)
            for s8m in (4, 5, 6, 7):
                scatter_desc(s8m, s8m % 4).wait()

        tables = (ego_h, l1_h, l2_h)
        for layer in range(N_LAYERS_):
            def zb(i, _):
                gbuf[i, pl.ds(0, 16)] = zeros16
                gbuf[i, pl.ds(16, 16)] = zeros16
                return _
            lax.fori_loop(0, zrows, zb, None)
            for z in range(n_zero):
                pltpu.sync_copy(gbuf.at[pl.ds(0, zrows)],
                                acc.at[pl.ds(row0 + z * zrows, zrows)])
            plsc.subcore_barrier()

            process_edges(tables[layer])
            plsc.subcore_barrier()

            if layer < N_LAYERS_ - 1:
                dst = l1_h if layer == 0 else l2_h
                pltpu.sync_copy(
                    acc.at[pl.ds(row0, rows_per_tec)],
                    dst.at[pl.ds(cid_off + row0, rows_per_tec)],
                )
            else:
                third = f32(1.0 / 3.0)
                db_o = drain_chunk
                dc_o = 2 * drain_chunk

                def dr(ch, _):
                    off = row0 + ch * drain_chunk
                    pltpu.sync_copy(acc.at[pl.ds(off, drain_chunk)],
                                    gbuf.at[pl.ds(0, drain_chunk)])
                    pltpu.sync_copy(l1_h.at[pl.ds(cid_off + off, drain_chunk)],
                                    gbuf.at[pl.ds(db_o, drain_chunk)])
                    pltpu.sync_copy(l2_h.at[pl.ds(cid_off + off, drain_chunk)],
                                    gbuf.at[pl.ds(dc_o, drain_chunk)])

                    def mr(r, _):
                        for h in (0, 16):
                            s = (gbuf[r, pl.ds(h, 16)]
                                 + gbuf[db_o + r, pl.ds(h, 16)]
                                 + gbuf[dc_o + r, pl.ds(h, 16)]) * third
                            gbuf[r, pl.ds(h, 16)] = s
                        return _
                    lax.fori_loop(0, drain_chunk, mr, None)
                    pltpu.sync_copy(gbuf.at[pl.ds(0, drain_chunk)],
                                    out_h.at[pl.ds(cid_off + off, drain_chunk)])
                    return _
                lax.fori_loop(0, n_drain, dr, None)
            plsc.subcore_barrier()

    return body(ego0, rows2d, cols2, vals)


def kernel(user_emb, item_emb, adj_row, adj_col, adj_val):
    n_user = user_emb.shape[0]
    n_item = item_emb.shape[0]
    n_nodes = n_user + n_item
    n_pad = -(-n_nodes // (NUM_TECS * 8)) * (NUM_TECS * 8)
    nnz = adj_row.shape[0]

    quant = NUM_TECS * CHUNK_E * 8
    nnz_pad = -(-nnz // quant) * quant
    chunks_per_tec = nnz_pad // (NUM_TECS * CHUNK_E)
    pad = nnz_pad - nnz
    rows_p = jnp.pad(adj_row, (0, pad))
    cols_p = jnp.pad(adj_col, (0, pad))
    vals_p = jnp.pad(adj_val, (0, pad))
    rows2d = rows_p.reshape(nnz_pad // CHUNK_E, CHUNK_E)
    cols2 = jnp.stack([cols_p, cols_p + jnp.int32(n_pad)])

    ego = jnp.concatenate([user_emb, item_emb], axis=0)
    rpad = n_pad - n_nodes
    lo = jnp.pad(ego[:, :32], ((0, rpad), (0, 0)))
    hi = jnp.pad(ego[:, 32:], ((0, rpad), (0, 0)))
    ego0 = jnp.concatenate([lo, hi], axis=0)

    out, _l1, _l2 = _propagate(n_pad, nnz_pad, chunks_per_tec,
                               ego0, rows2d, cols2, vals_p)
    all_emb = jnp.concatenate([out[:n_nodes], out[n_pad:n_pad + n_nodes]],
                              axis=1)
    return (all_emb[:n_user], all_emb[n_user:])
